# Initial kernel scaffold; baseline (speedup 1.0000x reference)
import jax
import jax.numpy as jnp
from jax.experimental import pallas as pl


def kernel(x, convW0, convb0, bng0, bnb0, convW1, convb1, bng1, bnb1, convW2, convb2, bng2, bnb2, fcW0, fcb0, fcW1, fcb1, outW, outb, edge_index, batch):
    raise NotImplementedError("write your pallas kernel here")


# speedup vs baseline: 10.2909x; 10.2909x over previous
import functools

import jax
import jax.numpy as jnp
from jax import lax
from jax.experimental import pallas as pl
from jax.experimental.pallas import tpu as pltpu
from jax.experimental.pallas import tpu_sc as plsc

N = 10000
E = 320000
G = 400
EPS = 1e-5

_NC = 2
_NS = 16
_NW = _NC * _NS
_EW = E // _NW
_CH = 80
_NCHUNK = _EW // _CH
_NP = 10240
_RPS = _NP // _NS

_HIGH = lax.Precision.HIGHEST


def _sc_mesh():
    return plsc.VectorSubcoreMesh(core_axis_name="c", subcore_axis_name="s")


def _sc_degree(dst, zeros16, ones_rows):
    @functools.partial(
        pl.kernel,
        out_type=jax.ShapeDtypeStruct((_NC, _NP, 128), jnp.float32),
        mesh=_sc_mesh(),
        scratch_types=[
            pltpu.VMEM((_CH,), jnp.int32),
            pltpu.VMEM((_CH, 128), jnp.float32),
            pltpu.VMEM_SHARED((_NP, 128), jnp.float32),
        ],
    )
    def deg_kernel(dst_hbm, z_hbm, ones_hbm, out_hbm, idx_v, ones_v, acc_sh):
        s_id = lax.axis_index("s")
        c_id = lax.axis_index("c")
        base = s_id * _RPS
        pltpu.sync_copy(z_hbm.at[pl.ds(base, _RPS)], acc_sh.at[pl.ds(base, _RPS)])
        pltpu.sync_copy(ones_hbm, ones_v)
        plsc.subcore_barrier()
        ebase = (s_id * _NC + c_id) * _EW

        @pl.loop(0, _NCHUNK)
        def _(i):
            pltpu.sync_copy(dst_hbm.at[pl.ds(ebase + i * _CH, _CH)], idx_v)
            pltpu.sync_copy(ones_v, acc_sh.at[idx_v], add=True)

        plsc.subcore_barrier()
        pltpu.sync_copy(acc_sh.at[pl.ds(base, _RPS)],
                        out_hbm.at[c_id, pl.ds(base, _RPS)])

    return deg_kernel(dst, zeros16, ones_rows)


def _sc_spmm(g, src, dst, zeros):
    D = g.shape[1]

    @functools.partial(
        pl.kernel,
        out_type=jax.ShapeDtypeStruct((_NC, _NP, D), jnp.float32),
        mesh=_sc_mesh(),
        scratch_types=[
            pltpu.VMEM((_CH,), jnp.int32),
            pltpu.VMEM((_CH,), jnp.int32),
            pltpu.VMEM((_CH, D), jnp.float32),
            pltpu.VMEM_SHARED((_NP, D), jnp.float32),
        ],
    )
    def spmm_kernel(g_hbm, src_hbm, dst_hbm, z_hbm, out_hbm,
                    sidx, didx, rows, acc_sh):
        s_id = lax.axis_index("s")
        c_id = lax.axis_index("c")
        base = s_id * _RPS
        pltpu.sync_copy(z_hbm.at[pl.ds(base, _RPS)], acc_sh.at[pl.ds(base, _RPS)])
        plsc.subcore_barrier()
        ebase = (s_id * _NC + c_id) * _EW

        @pl.loop(0, _NCHUNK)
        def _(i):
            off = ebase + i * _CH
            pltpu.sync_copy(src_hbm.at[pl.ds(off, _CH)], sidx)
            pltpu.sync_copy(dst_hbm.at[pl.ds(off, _CH)], didx)
            pltpu.sync_copy(g_hbm.at[sidx], rows)
            pltpu.sync_copy(rows, acc_sh.at[didx], add=True)

        plsc.subcore_barrier()
        pltpu.sync_copy(acc_sh.at[pl.ds(base, _RPS)],
                        out_hbm.at[c_id, pl.ds(base, _RPS)])

    return spmm_kernel(g, src, dst, zeros)


def _tc_matmul(x, w):
    def body(x_ref, w_ref, o_ref):
        o_ref[...] = jnp.dot(x_ref[...], w_ref[...],
                             preferred_element_type=jnp.float32)

    return pl.pallas_call(
        body,
        out_shape=jax.ShapeDtypeStruct((x.shape[0], w.shape[1]), jnp.float32),
    )(x, w)


def _tc_prep(degp, hw):
    def body(degp_ref, hw_ref, dinv_ref, g_ref):
        deg = degp_ref[0, :, 0:1] + degp_ref[1, :, 0:1] + 1.0
        dinv = 1.0 / jnp.sqrt(deg)
        dinv_ref[...] = jnp.broadcast_to(dinv, (N, 128))
        g_ref[...] = hw_ref[...] * dinv

    return pl.pallas_call(
        body,
        out_shape=(
            jax.ShapeDtypeStruct((N, 128), jnp.float32),
            jax.ShapeDtypeStruct((N, 128), jnp.float32),
        ),
    )(degp, hw)


def _bn_relu(pre, gamma, beta):
    m = jnp.mean(pre, axis=0, keepdims=True)
    c = pre - m
    v = jnp.mean(c * c, axis=0, keepdims=True)
    h = gamma * c / jnp.sqrt(v + EPS) + beta
    return jnp.maximum(h, 0.0)


def _tc_layer(accA, accB, g, dinv, b, gamma, beta, wn):
    Din = g.shape[1]
    Dout = wn.shape[1]

    def body(aA, aB, g_ref, dv_ref, b_ref, ga_ref, be_ref, w_ref, o_ref):
        dv = dv_ref[:, :Din]
        pre = dv * (aA[...] + aB[...] + g_ref[...]) + b_ref[...]
        h = _bn_relu(pre, ga_ref[...], be_ref[...])
        o_ref[...] = dv_ref[:, :Dout] * jnp.dot(
            h, w_ref[...], preferred_element_type=jnp.float32)

    return pl.pallas_call(
        body,
        out_shape=jax.ShapeDtypeStruct((N, Dout), jnp.float32),
    )(accA, accB, g, dinv, b.reshape(1, -1), gamma.reshape(1, -1),
      beta.reshape(1, -1), wn)


def _tc_layer_last(accA, accB, g, dinv, b, gamma, beta):
    Din = g.shape[1]

    def body(aA, aB, g_ref, dv_ref, b_ref, ga_ref, be_ref, o_ref):
        dv = dv_ref[:, :Din]
        pre = dv * (aA[...] + aB[...] + g_ref[...]) + b_ref[...]
        o_ref[...] = _bn_relu(pre, ga_ref[...], be_ref[...])

    return pl.pallas_call(
        body,
        out_shape=jax.ShapeDtypeStruct((N, Din), jnp.float32),
    )(accA, accB, g, dinv, b.reshape(1, -1), gamma.reshape(1, -1),
      beta.reshape(1, -1))


_GB = 80


def _tc_pool_head(batch2, h, fcW0, fcb0, fcW1, fcb1, outW, outb):
    def body(b_ref, h_ref, w0, b0, w1, b1, w2, b2, o_ref):
        g0 = pl.program_id(0) * _GB
        gids = g0 + lax.broadcasted_iota(jnp.int32, (_GB, N), 0)
        oh = (b_ref[...] == gids).astype(jnp.float32)
        counts = jnp.sum(oh, axis=1, keepdims=True)
        sums = jnp.dot(oh, h_ref[...], preferred_element_type=jnp.float32,
                       precision=_HIGH)
        pooled = sums / jnp.maximum(counts, 1.0)
        p = jnp.maximum(jnp.dot(pooled, w0[...],
                                preferred_element_type=jnp.float32) + b0[...], 0.0)
        p = jnp.maximum(jnp.dot(p, w1[...],
                                preferred_element_type=jnp.float32) + b1[...], 0.0)
        o_ref[...] = jnp.dot(p, w2[...],
                             preferred_element_type=jnp.float32) + b2[...]

    full = lambda shape: pl.BlockSpec(shape, lambda i: tuple(0 for _ in shape))
    return pl.pallas_call(
        body,
        grid=(G // _GB,),
        in_specs=[
            full((1, N)),
            full((N, 64)),
            full((64, 128)),
            full((1, 128)),
            full((128, 64)),
            full((1, 64)),
            full((64, 2)),
            full((1, 2)),
        ],
        out_specs=pl.BlockSpec((_GB, 2), lambda i: (i, 0)),
        out_shape=jax.ShapeDtypeStruct((G, 2), jnp.float32),
    )(batch2, h, fcW0, fcb0.reshape(1, -1), fcW1, fcb1.reshape(1, -1),
      outW, outb.reshape(1, -1))


def kernel(x, convW0, convb0, bng0, bnb0, convW1, convb1, bng1, bnb1,
           convW2, convb2, bng2, bnb2, fcW0, fcb0, fcW1, fcb1, outW, outb,
           edge_index, batch):
    src = edge_index[0]
    dst = edge_index[1]

    ones_rows = jnp.ones((_CH, 128), jnp.float32)
    z128 = jnp.zeros((_NP, 128), jnp.float32)

    degp = _sc_degree(dst, z128, ones_rows)
    hw0 = _tc_matmul(x, convW0)
    dinv, g0 = _tc_prep(degp[:, :N], hw0)

    acc0 = _sc_spmm(g0, src, dst, z128)
    g1 = _tc_layer(acc0[0, :N], acc0[1, :N], g0, dinv, convb0, bng0, bnb0, convW1)

    acc1 = _sc_spmm(g1, src, dst, z128)
    g2 = _tc_layer(acc1[0, :N], acc1[1, :N], g1, dinv, convb1, bng1, bnb1, convW2)

    g2p = jnp.pad(g2, ((0, 0), (0, 64)))
    acc2 = _sc_spmm(g2p, src, dst, z128)
    h3 = _tc_layer_last(acc2[0, :N, :64], acc2[1, :N, :64], g2, dinv,
                        convb2, bng2, bnb2)

    return _tc_pool_head(batch.reshape(1, N), h3, fcW0, fcb0, fcW1, fcb1,
                         outW, outb)


# speedup vs baseline: 16.8037x; 1.6329x over previous
import functools

import jax
import jax.numpy as jnp
from jax import lax
from jax.experimental import pallas as pl
from jax.experimental.pallas import tpu as pltpu
from jax.experimental.pallas import tpu_sc as plsc

N = 10000
E = 320000
G = 400
EPS = 1e-5

_NC = 2
_NS = 16
_NW = _NC * _NS
_EW = E // _NW
_CH = 80
_NCHUNK = _EW // _CH
_NP = 10240
_RPS = _NP // _NS

_HIGH = lax.Precision.HIGHEST


def _sc_mesh():
    return plsc.VectorSubcoreMesh(core_axis_name="c", subcore_axis_name="s")


def _sc_degree(dst, zeros16, ones_rows):
    @functools.partial(
        pl.kernel,
        out_type=jax.ShapeDtypeStruct((_NC, _NP, 128), jnp.float32),
        mesh=_sc_mesh(),
        scratch_types=[
            pltpu.VMEM((_CH,), jnp.int32),
            pltpu.VMEM((_CH, 128), jnp.float32),
            pltpu.VMEM_SHARED((_NP, 128), jnp.float32),
        ],
    )
    def deg_kernel(dst_hbm, z_hbm, ones_hbm, out_hbm, idx_v, ones_v, acc_sh):
        s_id = lax.axis_index("s")
        c_id = lax.axis_index("c")
        base = s_id * _RPS
        pltpu.sync_copy(z_hbm.at[pl.ds(base, _RPS)], acc_sh.at[pl.ds(base, _RPS)])
        pltpu.sync_copy(ones_hbm, ones_v)
        plsc.subcore_barrier()
        ebase = (s_id * _NC + c_id) * _EW

        @pl.loop(0, _NCHUNK)
        def _(i):
            pltpu.sync_copy(dst_hbm.at[pl.ds(ebase + i * _CH, _CH)], idx_v)
            pltpu.sync_copy(ones_v, acc_sh.at[idx_v], add=True)

        plsc.subcore_barrier()
        pltpu.sync_copy(acc_sh.at[pl.ds(base, _RPS)],
                        out_hbm.at[c_id, pl.ds(base, _RPS)])

    return deg_kernel(dst, zeros16, ones_rows)


def _sc_spmm(g, src, dst, zeros):
    @functools.partial(
        pl.kernel,
        out_type=jax.ShapeDtypeStruct((_NC, _NP, 128), jnp.float32),
        mesh=_sc_mesh(),
        scratch_types=[
            pltpu.VMEM((_EW,), jnp.int32),
            pltpu.VMEM((_NCHUNK, _CH), jnp.int32),
            pltpu.VMEM((_CH, 128), jnp.float32),
            pltpu.VMEM((_CH, 128), jnp.float32),
            pltpu.VMEM_SHARED((_NP, 128), jnp.float32),
            pltpu.SemaphoreType.DMA,
            pltpu.SemaphoreType.DMA,
            pltpu.SemaphoreType.DMA,
            pltpu.SemaphoreType.DMA,
            pltpu.SemaphoreType.DMA,
        ],
    )
    def spmm_kernel(g_hbm, src_hbm, dst_hbm, z_hbm, out_hbm,
                    sidx, didx, rowsA, rowsB, acc_sh, isem, gsA, gsB, ssA, ssB):
        s_id = lax.axis_index("s")
        c_id = lax.axis_index("c")
        base = s_id * _RPS
        ebase = (s_id * _NC + c_id) * _EW

        pltpu.async_copy(z_hbm.at[pl.ds(base, _RPS)],
                         acc_sh.at[pl.ds(base, _RPS)], isem)
        pltpu.async_copy(src_hbm.at[pl.ds(ebase, _EW)], sidx, isem)

        @pl.loop(0, _NCHUNK)
        def _(i):
            pltpu.async_copy(dst_hbm.at[pl.ds(ebase + i * _CH, _CH)],
                             didx.at[i], isem)

        pltpu.make_async_copy(z_hbm.at[pl.ds(base, _RPS)],
                              acc_sh.at[pl.ds(base, _RPS)], isem).wait()
        pltpu.make_async_copy(src_hbm.at[pl.ds(ebase, _EW)], sidx, isem).wait()

        @pl.loop(0, _NCHUNK)
        def _(i):
            pltpu.make_async_copy(dst_hbm.at[pl.ds(ebase, _CH)],
                                  didx.at[0], isem).wait()

        plsc.subcore_barrier()

        def start_gather(i, rows, sem):
            pltpu.async_copy(g_hbm.at[sidx.at[pl.ds(i * _CH, _CH)]], rows, sem)

        def wait_gather(rows, sem):
            pltpu.make_async_copy(g_hbm.at[pl.ds(0, _CH)], rows, sem).wait()

        def start_scatter(i, rows, sem):
            pltpu.async_copy(rows, acc_sh.at[didx.at[i]], sem, add=True)

        def wait_scatter(rows, sem):
            pltpu.make_async_copy(rows, acc_sh.at[pl.ds(0, _CH)], sem).wait()

        start_gather(0, rowsA, gsA)

        @pl.loop(0, (_NCHUNK - 1) // 2)
        def _(j):
            i = j * 2
            wait_gather(rowsA, gsA)

            @pl.when(j > 0)
            def _():
                wait_scatter(rowsB, ssB)

            start_gather(i + 1, rowsB, gsB)
            start_scatter(i, rowsA, ssA)

            wait_gather(rowsB, gsB)
            wait_scatter(rowsA, ssA)
            start_gather(i + 2, rowsA, gsA)
            start_scatter(i + 1, rowsB, ssB)

        wait_gather(rowsA, gsA)
        wait_scatter(rowsB, ssB)
        start_scatter(_NCHUNK - 1, rowsA, ssA)
        wait_scatter(rowsA, ssA)

        plsc.subcore_barrier()
        pltpu.sync_copy(acc_sh.at[pl.ds(base, _RPS)],
                        out_hbm.at[c_id, pl.ds(base, _RPS)])

    return spmm_kernel(g, src, dst, zeros)


def _tc_matmul(x, w):
    def body(x_ref, w_ref, o_ref):
        o_ref[...] = jnp.dot(x_ref[...], w_ref[...],
                             preferred_element_type=jnp.float32)

    return pl.pallas_call(
        body,
        out_shape=jax.ShapeDtypeStruct((x.shape[0], w.shape[1]), jnp.float32),
    )(x, w)


def _tc_prep(degp, hw):
    def body(degp_ref, hw_ref, dinv_ref, g_ref):
        deg = degp_ref[0, :, 0:1] + degp_ref[1, :, 0:1] + 1.0
        dinv = 1.0 / jnp.sqrt(deg)
        dinv_ref[...] = jnp.broadcast_to(dinv, (N, 128))
        g_ref[...] = hw_ref[...] * dinv

    return pl.pallas_call(
        body,
        out_shape=(
            jax.ShapeDtypeStruct((N, 128), jnp.float32),
            jax.ShapeDtypeStruct((N, 128), jnp.float32),
        ),
    )(degp, hw)


def _bn_relu(pre, gamma, beta):
    m = jnp.mean(pre, axis=0, keepdims=True)
    c = pre - m
    v = jnp.mean(c * c, axis=0, keepdims=True)
    h = gamma * c / jnp.sqrt(v + EPS) + beta
    return jnp.maximum(h, 0.0)


def _tc_layer(accA, accB, g, dinv, b, gamma, beta, wn):
    Din = g.shape[1]
    Dout = wn.shape[1]

    def body(aA, aB, g_ref, dv_ref, b_ref, ga_ref, be_ref, w_ref, o_ref):
        dv = dv_ref[:, :Din]
        pre = dv * (aA[...] + aB[...] + g_ref[...]) + b_ref[...]
        h = _bn_relu(pre, ga_ref[...], be_ref[...])
        o_ref[...] = dv_ref[:, :Dout] * jnp.dot(
            h, w_ref[...], preferred_element_type=jnp.float32)

    return pl.pallas_call(
        body,
        out_shape=jax.ShapeDtypeStruct((N, Dout), jnp.float32),
    )(accA, accB, g, dinv, b.reshape(1, -1), gamma.reshape(1, -1),
      beta.reshape(1, -1), wn)


def _tc_layer_last(accA, accB, g, dinv, b, gamma, beta):
    Din = g.shape[1]

    def body(aA, aB, g_ref, dv_ref, b_ref, ga_ref, be_ref, o_ref):
        dv = dv_ref[:, :Din]
        pre = dv * (aA[...] + aB[...] + g_ref[...]) + b_ref[...]
        o_ref[...] = _bn_relu(pre, ga_ref[...], be_ref[...])

    return pl.pallas_call(
        body,
        out_shape=jax.ShapeDtypeStruct((N, Din), jnp.float32),
    )(accA, accB, g, dinv, b.reshape(1, -1), gamma.reshape(1, -1),
      beta.reshape(1, -1))


_GB = 80


def _tc_pool_head(batch2, h, fcW0, fcb0, fcW1, fcb1, outW, outb):
    def body(b_ref, h_ref, w0, b0, w1, b1, w2, b2, o_ref):
        g0 = pl.program_id(0) * _GB
        gids = g0 + lax.broadcasted_iota(jnp.int32, (_GB, N), 0)
        oh = (b_ref[...] == gids).astype(jnp.float32)
        counts = jnp.sum(oh, axis=1, keepdims=True)
        sums = jnp.dot(oh, h_ref[...], preferred_element_type=jnp.float32,
                       precision=_HIGH)
        pooled = sums / jnp.maximum(counts, 1.0)
        p = jnp.maximum(jnp.dot(pooled, w0[...],
                                preferred_element_type=jnp.float32) + b0[...], 0.0)
        p = jnp.maximum(jnp.dot(p, w1[...],
                                preferred_element_type=jnp.float32) + b1[...], 0.0)
        o_ref[...] = jnp.dot(p, w2[...],
                             preferred_element_type=jnp.float32) + b2[...]

    full = lambda shape: pl.BlockSpec(shape, lambda i: tuple(0 for _ in shape))
    return pl.pallas_call(
        body,
        grid=(G // _GB,),
        in_specs=[
            full((1, N)),
            full((N, 64)),
            full((64, 128)),
            full((1, 128)),
            full((128, 64)),
            full((1, 64)),
            full((64, 2)),
            full((1, 2)),
        ],
        out_specs=pl.BlockSpec((_GB, 2), lambda i: (i, 0)),
        out_shape=jax.ShapeDtypeStruct((G, 2), jnp.float32),
    )(batch2, h, fcW0, fcb0.reshape(1, -1), fcW1, fcb1.reshape(1, -1),
      outW, outb.reshape(1, -1))


def kernel(x, convW0, convb0, bng0, bnb0, convW1, convb1, bng1, bnb1,
           convW2, convb2, bng2, bnb2, fcW0, fcb0, fcW1, fcb1, outW, outb,
           edge_index, batch):
    src = edge_index[0]
    dst = edge_index[1]

    ones_rows = jnp.ones((_CH, 128), jnp.float32)
    z128 = jnp.zeros((_NP, 128), jnp.float32)

    degp = _sc_degree(dst, z128, ones_rows)
    hw0 = _tc_matmul(x, convW0)
    dinv, g0 = _tc_prep(degp[:, :N], hw0)

    acc0 = _sc_spmm(g0, src, dst, z128)
    g1 = _tc_layer(acc0[0, :N], acc0[1, :N], g0, dinv, convb0, bng0, bnb0, convW1)

    acc1 = _sc_spmm(g1, src, dst, z128)
    g2 = _tc_layer(acc1[0, :N], acc1[1, :N], g1, dinv, convb1, bng1, bnb1, convW2)

    g2p = jnp.pad(g2, ((0, 0), (0, 64)))
    acc2 = _sc_spmm(g2p, src, dst, z128)
    h3 = _tc_layer_last(acc2[0, :N, :64], acc2[1, :N, :64], g2, dinv,
                        convb2, bng2, bnb2)

    return _tc_pool_head(batch.reshape(1, N), h3, fcW0, fcb0, fcW1, fcb1,
                         outW, outb)


# speedup vs baseline: 19.8865x; 1.1835x over previous
import dataclasses
import functools

import jax
import jax.numpy as jnp
from jax import lax
from jax.experimental import pallas as pl
from jax.experimental.pallas import tpu as pltpu
from jax.experimental.pallas import tpu_sc as plsc

N = 10000
E = 320000
G = 400
EPS = 1e-5

_NC = 2
_NS = 16
_NW = _NC * _NS
_EW = E // _NW
_CH = 80
_NCHUNK = _EW // _CH
_NP = 10240
_RPS = _NP // _NS

_HIGH = lax.Precision.HIGHEST


def _sc_mesh():
    return plsc.VectorSubcoreMesh(core_axis_name="c", subcore_axis_name="s")


def _sc_compiler_params():
    cp = pltpu.CompilerParams()
    if "needs_layout_passes" in pltpu.CompilerParams.__dataclass_fields__:
        cp = dataclasses.replace(cp, needs_layout_passes=False)
    return cp


def _sc_degree(dst, zeros):
    @functools.partial(
        pl.kernel,
        out_type=jax.ShapeDtypeStruct((_NC * _NP * 128,), jnp.float32),
        mesh=_sc_mesh(),
        compiler_params=_sc_compiler_params(),
        scratch_types=[
            pltpu.VMEM((_EW,), jnp.int32),
            pltpu.VMEM((80, 128), jnp.float32),
            pltpu.VMEM((80,), jnp.int32),
            pltpu.VMEM((5, 128), jnp.float32),
            pltpu.VMEM((80 * 128,), jnp.float32),
            pltpu.VMEM_SHARED((_NP, 128), jnp.float32),
            pltpu.SemaphoreType.DMA,
        ],
    )
    def deg_kernel(dst_hbm, z_hbm, out_hbm, dsts, acc2d, ident, pbuf, wbuf,
                   acc_sh, sem):
        s_id = lax.axis_index("s")
        c_id = lax.axis_index("c")
        base = s_id * _RPS
        ebase = (s_id * _NC + c_id) * _EW

        pltpu.async_copy(dst_hbm.at[pl.ds(ebase, _EW)], dsts, sem)

        @pl.when(s_id == 0)
        def _():
            pltpu.sync_copy(z_hbm.at[pl.ds(0, 80)], acc_sh.at[pl.ds(0, 80)])

        zero16 = jnp.zeros((16,), jnp.float32)
        iota16 = lax.iota(jnp.int32, 16)

        @pl.loop(0, 80)
        def _(r):
            @pl.loop(0, 8)
            def _(c):
                acc2d[r, pl.ds(c * 16, 16)] = zero16

        @pl.loop(0, 5)
        def _(j):
            ident[pl.ds(j * 16, 16)] = j * 16 + iota16

        pltpu.make_async_copy(dst_hbm.at[pl.ds(ebase, _EW)], dsts, sem).wait()

        ones16 = jnp.ones((16,), jnp.float32)

        @pl.loop(0, _EW // 16)
        def _(k):
            idx = dsts[pl.ds(k * 16, 16)]
            plsc.addupdate_scatter(
                acc2d,
                [lax.shift_right_logical(idx, 7), lax.bitwise_and(idx, 127)],
                ones16)

        plsc.subcore_barrier()
        pltpu.async_copy(acc2d, acc_sh.at[ident], sem, add=True)
        pltpu.make_async_copy(acc2d, acc_sh.at[pl.ds(0, 80)], sem).wait()
        plsc.subcore_barrier()

        pltpu.sync_copy(acc_sh.at[pl.ds(5 * s_id, 5)], pbuf)
        off = c_id * (_NP * 128) + base * 128

        @pl.loop(0, _RPS // 80)
        def _(b):
            @pl.loop(0, 5)
            def _(m):
                j = b * 80 + m * 16 + iota16
                vals = plsc.load_gather(
                    pbuf,
                    [lax.shift_right_logical(j, 7), lax.bitwise_and(j, 127)])
                plsc.store_scatter(wbuf, [(m * 16 + iota16) * 128], vals)

            pltpu.sync_copy(wbuf, out_hbm.at[pl.ds(off + b * 80 * 128,
                                                   80 * 128)])

    return deg_kernel(dst, zeros).reshape(_NC, _NP, 128)


def _sc_spmm(g, src, dst, zeros):
    @functools.partial(
        pl.kernel,
        out_type=jax.ShapeDtypeStruct((_NC, _NP, 128), jnp.float32),
        mesh=_sc_mesh(),
        scratch_types=[
            pltpu.VMEM((_EW,), jnp.int32),
            pltpu.VMEM((_NCHUNK, _CH), jnp.int32),
            pltpu.VMEM((_CH, 128), jnp.float32),
            pltpu.VMEM((_CH, 128), jnp.float32),
            pltpu.VMEM_SHARED((_NP, 128), jnp.float32),
            pltpu.SemaphoreType.DMA,
            pltpu.SemaphoreType.DMA,
            pltpu.SemaphoreType.DMA,
            pltpu.SemaphoreType.DMA,
            pltpu.SemaphoreType.DMA,
        ],
    )
    def spmm_kernel(g_hbm, src_hbm, dst_hbm, z_hbm, out_hbm,
                    sidx, didx, rowsA, rowsB, acc_sh, isem, gsA, gsB, ssA, ssB):
        s_id = lax.axis_index("s")
        c_id = lax.axis_index("c")
        base = s_id * _RPS
        ebase = (s_id * _NC + c_id) * _EW

        pltpu.async_copy(z_hbm.at[pl.ds(base, _RPS)],
                         acc_sh.at[pl.ds(base, _RPS)], isem)
        pltpu.async_copy(src_hbm.at[pl.ds(ebase, _EW)], sidx, isem)

        @pl.loop(0, _NCHUNK)
        def _(i):
            pltpu.async_copy(dst_hbm.at[pl.ds(ebase + i * _CH, _CH)],
                             didx.at[i], isem)

        pltpu.make_async_copy(z_hbm.at[pl.ds(base, _RPS)],
                              acc_sh.at[pl.ds(base, _RPS)], isem).wait()
        pltpu.make_async_copy(src_hbm.at[pl.ds(ebase, _EW)], sidx, isem).wait()

        @pl.loop(0, _NCHUNK)
        def _(i):
            pltpu.make_async_copy(dst_hbm.at[pl.ds(ebase, _CH)],
                                  didx.at[0], isem).wait()

        plsc.subcore_barrier()

        def start_gather(i, rows, sem):
            pltpu.async_copy(g_hbm.at[sidx.at[pl.ds(i * _CH, _CH)]], rows, sem)

        def wait_gather(rows, sem):
            pltpu.make_async_copy(g_hbm.at[pl.ds(0, _CH)], rows, sem).wait()

        def start_scatter(i, rows, sem):
            pltpu.async_copy(rows, acc_sh.at[didx.at[i]], sem, add=True)

        def wait_scatter(rows, sem):
            pltpu.make_async_copy(rows, acc_sh.at[pl.ds(0, _CH)], sem).wait()

        start_gather(0, rowsA, gsA)

        @pl.loop(0, (_NCHUNK - 1) // 2)
        def _(j):
            i = j * 2
            wait_gather(rowsA, gsA)

            @pl.when(j > 0)
            def _():
                wait_scatter(rowsB, ssB)

            start_gather(i + 1, rowsB, gsB)
            start_scatter(i, rowsA, ssA)

            wait_gather(rowsB, gsB)
            wait_scatter(rowsA, ssA)
            start_gather(i + 2, rowsA, gsA)
            start_scatter(i + 1, rowsB, ssB)

        wait_gather(rowsA, gsA)
        wait_scatter(rowsB, ssB)
        start_scatter(_NCHUNK - 1, rowsA, ssA)
        wait_scatter(rowsA, ssA)

        plsc.subcore_barrier()
        pltpu.sync_copy(acc_sh.at[pl.ds(base, _RPS)],
                        out_hbm.at[c_id, pl.ds(base, _RPS)])

    return spmm_kernel(g, src, dst, zeros)


def _tc_matmul(x, w):
    def body(x_ref, w_ref, o_ref):
        o_ref[...] = jnp.dot(x_ref[...], w_ref[...],
                             preferred_element_type=jnp.float32)

    return pl.pallas_call(
        body,
        out_shape=jax.ShapeDtypeStruct((x.shape[0], w.shape[1]), jnp.float32),
    )(x, w)


def _tc_prep(degp, hw):
    def body(degp_ref, hw_ref, dinv_ref, g_ref):
        deg = degp_ref[0, :, 0:1] + degp_ref[1, :, 0:1] + 1.0
        dinv = 1.0 / jnp.sqrt(deg)
        dinv_ref[...] = jnp.broadcast_to(dinv, (N, 128))
        g_ref[...] = hw_ref[...] * dinv

    return pl.pallas_call(
        body,
        out_shape=(
            jax.ShapeDtypeStruct((N, 128), jnp.float32),
            jax.ShapeDtypeStruct((N, 128), jnp.float32),
        ),
    )(degp, hw)


def _bn_relu(pre, gamma, beta):
    m = jnp.mean(pre, axis=0, keepdims=True)
    c = pre - m
    v = jnp.mean(c * c, axis=0, keepdims=True)
    h = gamma * c / jnp.sqrt(v + EPS) + beta
    return jnp.maximum(h, 0.0)


def _tc_layer(accA, accB, g, dinv, b, gamma, beta, wn):
    Din = g.shape[1]
    Dout = wn.shape[1]

    def body(aA, aB, g_ref, dv_ref, b_ref, ga_ref, be_ref, w_ref, o_ref):
        dv = dv_ref[:, :Din]
        pre = dv * (aA[...] + aB[...] + g_ref[...]) + b_ref[...]
        h = _bn_relu(pre, ga_ref[...], be_ref[...])
        o_ref[...] = dv_ref[:, :Dout] * jnp.dot(
            h, w_ref[...], preferred_element_type=jnp.float32)

    return pl.pallas_call(
        body,
        out_shape=jax.ShapeDtypeStruct((N, Dout), jnp.float32),
    )(accA, accB, g, dinv, b.reshape(1, -1), gamma.reshape(1, -1),
      beta.reshape(1, -1), wn)


def _tc_layer_last(accA, accB, g, dinv, b, gamma, beta):
    Din = g.shape[1]

    def body(aA, aB, g_ref, dv_ref, b_ref, ga_ref, be_ref, o_ref):
        dv = dv_ref[:, :Din]
        pre = dv * (aA[...] + aB[...] + g_ref[...]) + b_ref[...]
        o_ref[...] = _bn_relu(pre, ga_ref[...], be_ref[...])

    return pl.pallas_call(
        body,
        out_shape=jax.ShapeDtypeStruct((N, Din), jnp.float32),
    )(accA, accB, g, dinv, b.reshape(1, -1), gamma.reshape(1, -1),
      beta.reshape(1, -1))


_GB = 80


def _tc_pool_head(batch2, h, fcW0, fcb0, fcW1, fcb1, outW, outb):
    def body(b_ref, h_ref, w0, b0, w1, b1, w2, b2, o_ref):
        g0 = pl.program_id(0) * _GB
        gids = g0 + lax.broadcasted_iota(jnp.int32, (_GB, N), 0)
        oh = (b_ref[...] == gids).astype(jnp.float32)
        counts = jnp.sum(oh, axis=1, keepdims=True)
        sums = jnp.dot(oh, h_ref[...], preferred_element_type=jnp.float32,
                       precision=_HIGH)
        pooled = sums / jnp.maximum(counts, 1.0)
        p = jnp.maximum(jnp.dot(pooled, w0[...],
                                preferred_element_type=jnp.float32) + b0[...], 0.0)
        p = jnp.maximum(jnp.dot(p, w1[...],
                                preferred_element_type=jnp.float32) + b1[...], 0.0)
        o_ref[...] = jnp.dot(p, w2[...],
                             preferred_element_type=jnp.float32) + b2[...]

    full = lambda shape: pl.BlockSpec(shape, lambda i: tuple(0 for _ in shape))
    return pl.pallas_call(
        body,
        grid=(G // _GB,),
        in_specs=[
            full((1, N)),
            full((N, 64)),
            full((64, 128)),
            full((1, 128)),
            full((128, 64)),
            full((1, 64)),
            full((64, 2)),
            full((1, 2)),
        ],
        out_specs=pl.BlockSpec((_GB, 2), lambda i: (i, 0)),
        out_shape=jax.ShapeDtypeStruct((G, 2), jnp.float32),
    )(batch2, h, fcW0, fcb0.reshape(1, -1), fcW1, fcb1.reshape(1, -1),
      outW, outb.reshape(1, -1))


def kernel(x, convW0, convb0, bng0, bnb0, convW1, convb1, bng1, bnb1,
           convW2, convb2, bng2, bnb2, fcW0, fcb0, fcW1, fcb1, outW, outb,
           edge_index, batch):
    src = edge_index[0]
    dst = edge_index[1]

    z128 = jnp.zeros((_NP, 128), jnp.float32)

    degp = _sc_degree(dst, z128)
    hw0 = _tc_matmul(x, convW0)
    dinv, g0 = _tc_prep(degp[:, :N], hw0)

    acc0 = _sc_spmm(g0, src, dst, z128)
    g1 = _tc_layer(acc0[0, :N], acc0[1, :N], g0, dinv, convb0, bng0, bnb0, convW1)

    acc1 = _sc_spmm(g1, src, dst, z128)
    g2 = _tc_layer(acc1[0, :N], acc1[1, :N], g1, dinv, convb1, bng1, bnb1, convW2)

    g2p = jnp.pad(g2, ((0, 0), (0, 64)))
    acc2 = _sc_spmm(g2p, src, dst, z128)
    h3 = _tc_layer_last(acc2[0, :N, :64], acc2[1, :N, :64], g2, dinv,
                        convb2, bng2, bnb2)

    return _tc_pool_head(batch.reshape(1, N), h3, fcW0, fcb0, fcW1, fcb1,
                         outW, outb)


# speedup vs baseline: 21.5885x; 1.0856x over previous
import dataclasses
import functools

import jax
import jax.numpy as jnp
from jax import lax
from jax.experimental import pallas as pl
from jax.experimental.pallas import tpu as pltpu
from jax.experimental.pallas import tpu_sc as plsc

N = 10000
E = 320000
G = 400
EPS = 1e-5

_NC = 2
_NS = 16
_NW = _NC * _NS
_EW = E // _NW
_CHS = 104
_NF = _EW // _CHS
_NP = 10240
_RPS = _NP // _NS

_HIGH = lax.Precision.HIGHEST


def _sc_mesh():
    return plsc.VectorSubcoreMesh(core_axis_name="c", subcore_axis_name="s")


def _sc_compiler_params():
    cp = pltpu.CompilerParams()
    if "needs_layout_passes" in pltpu.CompilerParams.__dataclass_fields__:
        cp = dataclasses.replace(cp, needs_layout_passes=False)
    return cp


def _sc_degree(dst, zeros):
    @functools.partial(
        pl.kernel,
        out_type=jax.ShapeDtypeStruct((_NC * _NP * 128,), jnp.float32),
        mesh=_sc_mesh(),
        compiler_params=_sc_compiler_params(),
        scratch_types=[
            pltpu.VMEM((_EW,), jnp.int32),
            pltpu.VMEM((80, 128), jnp.float32),
            pltpu.VMEM((5, 128), jnp.float32),
            pltpu.VMEM((80 * 128,), jnp.float32),
            pltpu.VMEM_SHARED((_NP, 128), jnp.float32),
            pltpu.SemaphoreType.DMA,
        ],
    )
    def deg_kernel(dst_hbm, z_hbm, out_hbm, dsts, acc2d, pbuf, wbuf,
                   acc_sh, sem):
        s_id = lax.axis_index("s")
        c_id = lax.axis_index("c")
        base = s_id * _RPS
        ebase = (s_id * _NC + c_id) * _EW

        pltpu.async_copy(dst_hbm.at[pl.ds(ebase, _EW)], dsts, sem)

        zero16 = jnp.zeros((16,), jnp.float32)
        iota16 = lax.iota(jnp.int32, 16)

        @pl.loop(0, 80)
        def _(r):
            @pl.loop(0, 8)
            def _(c):
                acc2d[r, pl.ds(c * 16, 16)] = zero16

        pltpu.make_async_copy(dst_hbm.at[pl.ds(ebase, _EW)], dsts, sem).wait()

        ones16 = jnp.ones((16,), jnp.float32)

        @pl.loop(0, _EW // 16)
        def _(k):
            idx = dsts[pl.ds(k * 16, 16)]
            plsc.addupdate_scatter(
                acc2d,
                [lax.shift_right_logical(idx, 7), lax.bitwise_and(idx, 127)],
                ones16)

        pltpu.sync_copy(acc2d, acc_sh.at[pl.ds(80 * s_id, 80)])
        plsc.subcore_barrier()

        @pl.loop(0, _NS)
        def _(t):
            pltpu.async_copy(acc_sh.at[pl.ds(80 * t + 5 * s_id, 5)],
                             acc2d.at[pl.ds(5 * t, 5)], sem)

        @pl.loop(0, _NS)
        def _(t):
            pltpu.make_async_copy(acc_sh.at[pl.ds(0, 5)],
                                  acc2d.at[pl.ds(0, 5)], sem).wait()

        @pl.loop(0, 5)
        def _(r):
            @pl.loop(0, 8)
            def _(c):
                sl = pl.ds(c * 16, 16)
                acc = acc2d[0 * 5 + r, sl]
                pbuf[r, sl] = acc

        @pl.loop(1, _NS)
        def _(t):
            @pl.loop(0, 5)
            def _(r):
                @pl.loop(0, 8)
                def _(c):
                    sl = pl.ds(c * 16, 16)
                    pbuf[r, sl] = pbuf[r, sl] + acc2d[t * 5 + r, sl]

        off = c_id * (_NP * 128) + base * 128

        @pl.loop(0, _RPS // 80)
        def _(b):
            @pl.loop(0, 5)
            def _(m):
                j = b * 80 + m * 16 + iota16
                vals = plsc.load_gather(
                    pbuf,
                    [lax.shift_right_logical(j, 7), lax.bitwise_and(j, 127)])
                plsc.store_scatter(wbuf, [(m * 16 + iota16) * 128], vals)

            pltpu.sync_copy(wbuf, out_hbm.at[pl.ds(off + b * 80 * 128,
                                                   80 * 128)])

    return deg_kernel(dst, zeros).reshape(_NC, _NP, 128)


def _sc_spmm(g, src, dst, zeros):
    @functools.partial(
        pl.kernel,
        out_type=jax.ShapeDtypeStruct((_NC, _NP, 128), jnp.float32),
        mesh=_sc_mesh(),
        scratch_types=[
            pltpu.VMEM((_EW,), jnp.int32),
            pltpu.VMEM((_NF, _CHS), jnp.int32),
            pltpu.VMEM((16,), jnp.int32),
            pltpu.VMEM((_CHS, 128), jnp.float32),
            pltpu.VMEM((_CHS, 128), jnp.float32),
            pltpu.VMEM_SHARED((_NP, 128), jnp.float32),
            pltpu.SemaphoreType.DMA,
            pltpu.SemaphoreType.DMA,
            pltpu.SemaphoreType.DMA,
            pltpu.SemaphoreType.DMA,
            pltpu.SemaphoreType.DMA,
        ],
    )
    def spmm_kernel(g_hbm, src_hbm, dst_hbm, z_hbm, out_hbm,
                    sidx, didx, didxT, rowsA, rowsB, acc_sh,
                    isem, gsA, gsB, ssA, ssB):
        s_id = lax.axis_index("s")
        c_id = lax.axis_index("c")
        base = s_id * _RPS
        ebase = (s_id * _NC + c_id) * _EW

        pltpu.async_copy(z_hbm.at[pl.ds(base, _RPS)],
                         acc_sh.at[pl.ds(base, _RPS)], isem)
        pltpu.async_copy(src_hbm.at[pl.ds(ebase, _EW)], sidx, isem)
        pltpu.async_copy(dst_hbm.at[pl.ds(ebase + _NF * _CHS, 16)],
                         didxT, isem)

        @pl.loop(0, _NF)
        def _(i):
            pltpu.async_copy(dst_hbm.at[pl.ds(ebase + i * _CHS, _CHS)],
                             didx.at[i], isem)

        pltpu.make_async_copy(z_hbm.at[pl.ds(base, _RPS)],
                              acc_sh.at[pl.ds(base, _RPS)], isem).wait()
        pltpu.make_async_copy(src_hbm.at[pl.ds(ebase, _EW)], sidx, isem).wait()
        pltpu.make_async_copy(dst_hbm.at[pl.ds(ebase, 16)], didxT, isem).wait()

        @pl.loop(0, _NF)
        def _(i):
            pltpu.make_async_copy(dst_hbm.at[pl.ds(ebase, _CHS)],
                                  didx.at[0], isem).wait()

        plsc.subcore_barrier()

        def start_gather(i, rows, sem):
            pltpu.async_copy(g_hbm.at[sidx.at[pl.ds(i * _CHS, _CHS)]], rows,
                             sem)

        def wait_gather(rows, sem):
            pltpu.make_async_copy(g_hbm.at[pl.ds(0, _CHS)], rows, sem).wait()

        def start_scatter(i, rows, sem):
            pltpu.async_copy(rows, acc_sh.at[didx.at[i]], sem, add=True)

        def wait_scatter(rows, sem):
            pltpu.make_async_copy(rows, acc_sh.at[pl.ds(0, _CHS)], sem).wait()

        start_gather(0, rowsA, gsA)

        @pl.loop(0, (_NF - 1) // 2)
        def _(j):
            i = j * 2
            wait_gather(rowsA, gsA)

            @pl.when(j > 0)
            def _():
                wait_scatter(rowsB, ssB)

            start_gather(i + 1, rowsB, gsB)
            start_scatter(i, rowsA, ssA)

            wait_gather(rowsB, gsB)
            wait_scatter(rowsA, ssA)
            start_gather(i + 2, rowsA, gsA)
            start_scatter(i + 1, rowsB, ssB)

        wait_gather(rowsA, gsA)
        wait_scatter(rowsB, ssB)
        start_gather(_NF - 1, rowsB, gsB)
        start_scatter(_NF - 2, rowsA, ssA)
        wait_gather(rowsB, gsB)
        start_scatter(_NF - 1, rowsB, ssB)
        wait_scatter(rowsA, ssA)

        rowsT = rowsA.at[pl.ds(0, 16)]
        pltpu.async_copy(g_hbm.at[sidx.at[pl.ds(_NF * _CHS, 16)]], rowsT, gsA)
        pltpu.make_async_copy(g_hbm.at[pl.ds(0, 16)], rowsT, gsA).wait()
        pltpu.sync_copy(rowsT, acc_sh.at[didxT], add=True)
        wait_scatter(rowsB, ssB)

        plsc.subcore_barrier()
        pltpu.sync_copy(acc_sh.at[pl.ds(base, _RPS)],
                        out_hbm.at[c_id, pl.ds(base, _RPS)])

    return spmm_kernel(g, src, dst, zeros)


def _tc_matmul(x, w):
    def body(x_ref, w_ref, o_ref):
        o_ref[...] = jnp.dot(x_ref[...], w_ref[...],
                             preferred_element_type=jnp.float32)

    return pl.pallas_call(
        body,
        out_shape=jax.ShapeDtypeStruct((x.shape[0], w.shape[1]), jnp.float32),
    )(x, w)


def _tc_prep(degp, hw):
    def body(degp_ref, hw_ref, dinv_ref, g_ref):
        deg = degp_ref[0, :, 0:1] + degp_ref[1, :, 0:1] + 1.0
        dinv = 1.0 / jnp.sqrt(deg)
        dinv_ref[...] = jnp.broadcast_to(dinv, (N, 128))
        g_ref[...] = hw_ref[...] * dinv

    return pl.pallas_call(
        body,
        out_shape=(
            jax.ShapeDtypeStruct((N, 128), jnp.float32),
            jax.ShapeDtypeStruct((N, 128), jnp.float32),
        ),
    )(degp, hw)


def _bn_relu(pre, gamma, beta):
    m = jnp.mean(pre, axis=0, keepdims=True)
    c = pre - m
    v = jnp.mean(c * c, axis=0, keepdims=True)
    h = gamma * c / jnp.sqrt(v + EPS) + beta
    return jnp.maximum(h, 0.0)


def _tc_layer(accA, accB, g, dinv, b, gamma, beta, wn):
    Din = g.shape[1]
    Dout = wn.shape[1]

    def body(aA, aB, g_ref, dv_ref, b_ref, ga_ref, be_ref, w_ref, o_ref):
        dv = dv_ref[:, :Din]
        pre = dv * (aA[...] + aB[...] + g_ref[...]) + b_ref[...]
        h = _bn_relu(pre, ga_ref[...], be_ref[...])
        o_ref[...] = dv_ref[:, :Dout] * jnp.dot(
            h, w_ref[...], preferred_element_type=jnp.float32)

    return pl.pallas_call(
        body,
        out_shape=jax.ShapeDtypeStruct((N, Dout), jnp.float32),
    )(accA, accB, g, dinv, b.reshape(1, -1), gamma.reshape(1, -1),
      beta.reshape(1, -1), wn)


def _tc_layer_last(accA, accB, g, dinv, b, gamma, beta):
    Din = g.shape[1]

    def body(aA, aB, g_ref, dv_ref, b_ref, ga_ref, be_ref, o_ref):
        dv = dv_ref[:, :Din]
        pre = dv * (aA[...] + aB[...] + g_ref[...]) + b_ref[...]
        o_ref[...] = _bn_relu(pre, ga_ref[...], be_ref[...])

    return pl.pallas_call(
        body,
        out_shape=jax.ShapeDtypeStruct((N, Din), jnp.float32),
    )(accA, accB, g, dinv, b.reshape(1, -1), gamma.reshape(1, -1),
      beta.reshape(1, -1))


_GB = 80


def _tc_pool_head(batch2, h, fcW0, fcb0, fcW1, fcb1, outW, outb):
    def body(b_ref, h_ref, w0, b0, w1, b1, w2, b2, o_ref):
        g0 = pl.program_id(0) * _GB
        gids = g0 + lax.broadcasted_iota(jnp.int32, (_GB, N), 0)
        oh = (b_ref[...] == gids).astype(jnp.float32)
        counts = jnp.sum(oh, axis=1, keepdims=True)
        sums = jnp.dot(oh, h_ref[...], preferred_element_type=jnp.float32,
                       precision=_HIGH)
        pooled = sums / jnp.maximum(counts, 1.0)
        p = jnp.maximum(jnp.dot(pooled, w0[...],
                                preferred_element_type=jnp.float32) + b0[...], 0.0)
        p = jnp.maximum(jnp.dot(p, w1[...],
                                preferred_element_type=jnp.float32) + b1[...], 0.0)
        o_ref[...] = jnp.dot(p, w2[...],
                             preferred_element_type=jnp.float32) + b2[...]

    full = lambda shape: pl.BlockSpec(shape, lambda i: tuple(0 for _ in shape))
    return pl.pallas_call(
        body,
        grid=(G // _GB,),
        in_specs=[
            full((1, N)),
            full((N, 64)),
            full((64, 128)),
            full((1, 128)),
            full((128, 64)),
            full((1, 64)),
            full((64, 2)),
            full((1, 2)),
        ],
        out_specs=pl.BlockSpec((_GB, 2), lambda i: (i, 0)),
        out_shape=jax.ShapeDtypeStruct((G, 2), jnp.float32),
    )(batch2, h, fcW0, fcb0.reshape(1, -1), fcW1, fcb1.reshape(1, -1),
      outW, outb.reshape(1, -1))


def kernel(x, convW0, convb0, bng0, bnb0, convW1, convb1, bng1, bnb1,
           convW2, convb2, bng2, bnb2, fcW0, fcb0, fcW1, fcb1, outW, outb,
           edge_index, batch):
    src = edge_index[0]
    dst = edge_index[1]

    z128 = jnp.zeros((_NP, 128), jnp.float32)

    degp = _sc_degree(dst, z128)
    hw0 = _tc_matmul(x, convW0)
    dinv, g0 = _tc_prep(degp[:, :N], hw0)

    acc0 = _sc_spmm(g0, src, dst, z128)
    g1 = _tc_layer(acc0[0, :N], acc0[1, :N], g0, dinv, convb0, bng0, bnb0, convW1)

    acc1 = _sc_spmm(g1, src, dst, z128)
    g2 = _tc_layer(acc1[0, :N], acc1[1, :N], g1, dinv, convb1, bng1, bnb1, convW2)

    g2p = jnp.pad(g2, ((0, 0), (0, 64)))
    acc2 = _sc_spmm(g2p, src, dst, z128)
    h3 = _tc_layer_last(acc2[0, :N, :64], acc2[1, :N, :64], g2, dinv,
                        convb2, bng2, bnb2)

    return _tc_pool_head(batch.reshape(1, N), h3, fcW0, fcb0, fcW1, fcb1,
                         outW, outb)


# speedup vs baseline: 22.8102x; 1.0566x over previous
import dataclasses
import functools

import jax
import jax.numpy as jnp
from jax import lax
from jax.experimental import pallas as pl
from jax.experimental.pallas import tpu as pltpu
from jax.experimental.pallas import tpu_sc as plsc

N = 10000
E = 320000
G = 400
EPS = 1e-5

_NC = 2
_NS = 16
_NW = _NC * _NS
_EW = E // _NW
_CHS = 104
_NF = _EW // _CHS
_NP = 10240
_RPS = _NP // _NS

_HIGH = lax.Precision.HIGHEST


def _sc_mesh():
    return plsc.VectorSubcoreMesh(core_axis_name="c", subcore_axis_name="s")


def _sc_compiler_params():
    cp = pltpu.CompilerParams()
    if "needs_layout_passes" in pltpu.CompilerParams.__dataclass_fields__:
        cp = dataclasses.replace(cp, needs_layout_passes=False)
    return cp


def _sc_degree(dst, zeros):
    @functools.partial(
        pl.kernel,
        out_type=jax.ShapeDtypeStruct((_NC * _NP * 128,), jnp.float32),
        mesh=_sc_mesh(),
        compiler_params=_sc_compiler_params(),
        scratch_types=[
            pltpu.VMEM((_EW,), jnp.int32),
            pltpu.VMEM((80, 128), jnp.float32),
            pltpu.VMEM((5, 128), jnp.float32),
            pltpu.VMEM((80 * 128,), jnp.float32),
            pltpu.VMEM_SHARED((_NP, 128), jnp.float32),
            pltpu.SemaphoreType.DMA,
        ],
    )
    def deg_kernel(dst_hbm, z_hbm, out_hbm, dsts, acc2d, pbuf, wbuf,
                   acc_sh, sem):
        s_id = lax.axis_index("s")
        c_id = lax.axis_index("c")
        base = s_id * _RPS
        ebase = (s_id * _NC + c_id) * _EW

        pltpu.async_copy(dst_hbm.at[pl.ds(ebase, _EW)], dsts, sem)

        zero16 = jnp.zeros((16,), jnp.float32)
        iota16 = lax.iota(jnp.int32, 16)

        @pl.loop(0, 80)
        def _(r):
            @pl.loop(0, 8)
            def _(c):
                acc2d[r, pl.ds(c * 16, 16)] = zero16

        pltpu.make_async_copy(dst_hbm.at[pl.ds(ebase, _EW)], dsts, sem).wait()

        ones16 = jnp.ones((16,), jnp.float32)

        @pl.loop(0, _EW // 16)
        def _(k):
            idx = dsts[pl.ds(k * 16, 16)]
            plsc.addupdate_scatter(
                acc2d,
                [lax.shift_right_logical(idx, 7), lax.bitwise_and(idx, 127)],
                ones16)

        pltpu.sync_copy(acc2d, acc_sh.at[pl.ds(80 * s_id, 80)])
        plsc.subcore_barrier()

        @pl.loop(0, _NS)
        def _(t):
            pltpu.async_copy(acc_sh.at[pl.ds(80 * t + 5 * s_id, 5)],
                             acc2d.at[pl.ds(5 * t, 5)], sem)

        @pl.loop(0, _NS)
        def _(t):
            pltpu.make_async_copy(acc_sh.at[pl.ds(0, 5)],
                                  acc2d.at[pl.ds(0, 5)], sem).wait()

        @pl.loop(0, 5)
        def _(r):
            @pl.loop(0, 8)
            def _(c):
                sl = pl.ds(c * 16, 16)
                acc = acc2d[0 * 5 + r, sl]
                pbuf[r, sl] = acc

        @pl.loop(1, _NS)
        def _(t):
            @pl.loop(0, 5)
            def _(r):
                @pl.loop(0, 8)
                def _(c):
                    sl = pl.ds(c * 16, 16)
                    pbuf[r, sl] = pbuf[r, sl] + acc2d[t * 5 + r, sl]

        off = c_id * (_NP * 128) + base * 128

        @pl.loop(0, _RPS // 80)
        def _(b):
            @pl.loop(0, 5)
            def _(m):
                j = b * 80 + m * 16 + iota16
                vals = plsc.load_gather(
                    pbuf,
                    [lax.shift_right_logical(j, 7), lax.bitwise_and(j, 127)])
                plsc.store_scatter(wbuf, [(m * 16 + iota16) * 128], vals)

            pltpu.sync_copy(wbuf, out_hbm.at[pl.ds(off + b * 80 * 128,
                                                   80 * 128)])

    return deg_kernel(dst, zeros).reshape(_NC, _NP, 128)


def _sc_spmm(g, src, dst, zeros):
    @functools.partial(
        pl.kernel,
        out_type=jax.ShapeDtypeStruct((_NC, _NP, 128), jnp.float32),
        mesh=_sc_mesh(),
        scratch_types=[
            pltpu.VMEM((_EW,), jnp.int32),
            pltpu.VMEM((_NF, _CHS), jnp.int32),
            pltpu.VMEM((16,), jnp.int32),
            pltpu.VMEM((_CHS, 128), jnp.float32),
            pltpu.VMEM((_CHS, 128), jnp.float32),
            pltpu.VMEM_SHARED((_NP, 128), jnp.float32),
            pltpu.SemaphoreType.DMA,
            pltpu.SemaphoreType.DMA,
            pltpu.SemaphoreType.DMA,
            pltpu.SemaphoreType.DMA,
            pltpu.SemaphoreType.DMA,
        ],
    )
    def spmm_kernel(g_hbm, src_hbm, dst_hbm, z_hbm, out_hbm,
                    sidx, didx, didxT, rowsA, rowsB, acc_sh,
                    isem, gsA, gsB, ssA, ssB):
        s_id = lax.axis_index("s")
        c_id = lax.axis_index("c")
        base = s_id * _RPS
        ebase = (s_id * _NC + c_id) * _EW

        pltpu.async_copy(z_hbm.at[pl.ds(base, _RPS)],
                         acc_sh.at[pl.ds(base, _RPS)], isem)
        pltpu.async_copy(src_hbm.at[pl.ds(ebase, _EW)], sidx, isem)
        pltpu.async_copy(dst_hbm.at[pl.ds(ebase + _NF * _CHS, 16)],
                         didxT, isem)

        @pl.loop(0, _NF)
        def _(i):
            pltpu.async_copy(dst_hbm.at[pl.ds(ebase + i * _CHS, _CHS)],
                             didx.at[i], isem)

        pltpu.make_async_copy(z_hbm.at[pl.ds(base, _RPS)],
                              acc_sh.at[pl.ds(base, _RPS)], isem).wait()
        pltpu.make_async_copy(src_hbm.at[pl.ds(ebase, _EW)], sidx, isem).wait()
        pltpu.make_async_copy(dst_hbm.at[pl.ds(ebase, 16)], didxT, isem).wait()

        @pl.loop(0, _NF)
        def _(i):
            pltpu.make_async_copy(dst_hbm.at[pl.ds(ebase, _CHS)],
                                  didx.at[0], isem).wait()

        plsc.subcore_barrier()

        def start_gather(i, rows, sem):
            pltpu.async_copy(g_hbm.at[sidx.at[pl.ds(i * _CHS, _CHS)]], rows,
                             sem)

        def wait_gather(rows, sem):
            pltpu.make_async_copy(g_hbm.at[pl.ds(0, _CHS)], rows, sem).wait()

        def start_scatter(i, rows, sem):
            pltpu.async_copy(rows, acc_sh.at[didx.at[i]], sem, add=True)

        def wait_scatter(rows, sem):
            pltpu.make_async_copy(rows, acc_sh.at[pl.ds(0, _CHS)], sem).wait()

        start_gather(0, rowsA, gsA)

        @pl.loop(0, (_NF - 1) // 2)
        def _(j):
            i = j * 2
            wait_gather(rowsA, gsA)

            @pl.when(j > 0)
            def _():
                wait_scatter(rowsB, ssB)

            start_gather(i + 1, rowsB, gsB)
            start_scatter(i, rowsA, ssA)

            wait_gather(rowsB, gsB)
            wait_scatter(rowsA, ssA)
            start_gather(i + 2, rowsA, gsA)
            start_scatter(i + 1, rowsB, ssB)

        wait_gather(rowsA, gsA)
        wait_scatter(rowsB, ssB)
        start_gather(_NF - 1, rowsB, gsB)
        start_scatter(_NF - 2, rowsA, ssA)
        wait_gather(rowsB, gsB)
        start_scatter(_NF - 1, rowsB, ssB)
        wait_scatter(rowsA, ssA)

        rowsT = rowsA.at[pl.ds(0, 16)]
        pltpu.async_copy(g_hbm.at[sidx.at[pl.ds(_NF * _CHS, 16)]], rowsT, gsA)
        pltpu.make_async_copy(g_hbm.at[pl.ds(0, 16)], rowsT, gsA).wait()
        pltpu.sync_copy(rowsT, acc_sh.at[didxT], add=True)
        wait_scatter(rowsB, ssB)

        plsc.subcore_barrier()
        pltpu.sync_copy(acc_sh.at[pl.ds(base, _RPS)],
                        out_hbm.at[c_id, pl.ds(base, _RPS)])

    return spmm_kernel(g, src, dst, zeros)


def _tc_matmul(x, w):
    def body(x_ref, w_ref, o_ref):
        o_ref[...] = jnp.dot(x_ref[...], w_ref[...],
                             preferred_element_type=jnp.float32)

    return pl.pallas_call(
        body,
        out_shape=jax.ShapeDtypeStruct((x.shape[0], w.shape[1]), jnp.float32),
    )(x, w)


def _tc_prep(degp, hw):
    def body(degp_ref, hw_ref, dinv_ref, g_ref):
        deg = degp_ref[0, :N, 0:1] + degp_ref[1, :N, 0:1] + 1.0
        dinv = 1.0 / jnp.sqrt(deg)
        dinv_ref[...] = jnp.broadcast_to(dinv, (N, 128))
        g_ref[...] = hw_ref[...] * dinv

    return pl.pallas_call(
        body,
        out_shape=(
            jax.ShapeDtypeStruct((N, 128), jnp.float32),
            jax.ShapeDtypeStruct((N, 128), jnp.float32),
        ),
    )(degp, hw)


def _bn_relu(pre, gamma, beta):
    m = jnp.mean(pre, axis=0, keepdims=True)
    c = pre - m
    v = jnp.mean(c * c, axis=0, keepdims=True)
    h = gamma * c / jnp.sqrt(v + EPS) + beta
    return jnp.maximum(h, 0.0)


def _tc_layer(acc, g, dinv, b, gamma, beta, wn, gdin=None, pad_out=False):
    Din = g.shape[1] if gdin is None else gdin

    def body(aA_aB, g_ref, dv_ref, b_ref, ga_ref, be_ref, *rest):
        dv = dv_ref[:, :Din]
        pre = dv * (aA_aB[0, :N, :Din] + aA_aB[1, :N, :Din]
                    + g_ref[:, :Din]) + b_ref[...]
        h = _bn_relu(pre, ga_ref[...], be_ref[...])
        if wn is None:
            rest[-1][...] = h
        else:
            w_ref, o_ref = rest
            out = dv_ref[:, :wn.shape[1]] * jnp.dot(
                h, w_ref[...], preferred_element_type=jnp.float32)
            if pad_out:
                o_ref[...] = jnp.concatenate(
                    [out, jnp.zeros((N, 128 - wn.shape[1]), jnp.float32)],
                    axis=1)
            else:
                o_ref[...] = out

    dout = Din if wn is None else (128 if pad_out else wn.shape[1])
    args = [acc, g, dinv, b.reshape(1, -1), gamma.reshape(1, -1),
            beta.reshape(1, -1)]
    if wn is not None:
        args.append(wn)
    return pl.pallas_call(
        body,
        out_shape=jax.ShapeDtypeStruct((N, dout), jnp.float32),
    )(*args)


_GB = 80


def _tc_pool_head(batch2, h, fcW0, fcb0, fcW1, fcb1, outW, outb):
    def body(b_ref, h_ref, w0, b0, w1, b1, w2, b2, o_ref):
        g0 = pl.program_id(0) * _GB
        gids = g0 + lax.broadcasted_iota(jnp.int32, (_GB, N), 0)
        oh = (b_ref[...] == gids).astype(jnp.float32)
        counts = jnp.sum(oh, axis=1, keepdims=True)
        sums = jnp.dot(oh, h_ref[...], preferred_element_type=jnp.float32,
                       precision=_HIGH)
        pooled = sums / jnp.maximum(counts, 1.0)
        p = jnp.maximum(jnp.dot(pooled, w0[...],
                                preferred_element_type=jnp.float32) + b0[...], 0.0)
        p = jnp.maximum(jnp.dot(p, w1[...],
                                preferred_element_type=jnp.float32) + b1[...], 0.0)
        o_ref[...] = jnp.dot(p, w2[...],
                             preferred_element_type=jnp.float32) + b2[...]

    full = lambda shape: pl.BlockSpec(shape, lambda i: tuple(0 for _ in shape))
    return pl.pallas_call(
        body,
        grid=(G // _GB,),
        in_specs=[
            full((1, N)),
            full((N, 64)),
            full((64, 128)),
            full((1, 128)),
            full((128, 64)),
            full((1, 64)),
            full((64, 2)),
            full((1, 2)),
        ],
        out_specs=pl.BlockSpec((_GB, 2), lambda i: (i, 0)),
        out_shape=jax.ShapeDtypeStruct((G, 2), jnp.float32),
    )(batch2, h, fcW0, fcb0.reshape(1, -1), fcW1, fcb1.reshape(1, -1),
      outW, outb.reshape(1, -1))


def kernel(x, convW0, convb0, bng0, bnb0, convW1, convb1, bng1, bnb1,
           convW2, convb2, bng2, bnb2, fcW0, fcb0, fcW1, fcb1, outW, outb,
           edge_index, batch):
    src = edge_index[0]
    dst = edge_index[1]

    z128 = jnp.zeros((_NP, 128), jnp.float32)

    degp = _sc_degree(dst, z128)
    hw0 = _tc_matmul(x, convW0)
    dinv, g0 = _tc_prep(degp, hw0)

    acc0 = _sc_spmm(g0, src, dst, z128)
    g1 = _tc_layer(acc0, g0, dinv, convb0, bng0, bnb0, convW1)

    acc1 = _sc_spmm(g1, src, dst, z128)
    g2p = _tc_layer(acc1, g1, dinv, convb1, bng1, bnb1, convW2, pad_out=True)

    acc2 = _sc_spmm(g2p, src, dst, z128)
    h3 = _tc_layer(acc2, g2p, dinv, convb2, bng2, bnb2, None, gdin=64)

    return _tc_pool_head(batch.reshape(1, N), h3, fcW0, fcb0, fcW1, fcb1,
                         outW, outb)
